# baseline (speedup 1.0000x reference)
import functools

import jax
import jax.numpy as jnp
from jax import lax
from jax.experimental import pallas as pl
from jax.experimental.pallas import tpu as pltpu
from jax.experimental.pallas import tpu_sc as plsc

EMB = 64
NUM_CORES = 2
NUM_SUBCORES = 16
NW = NUM_CORES * NUM_SUBCORES
LANES = 16
BLK = 128
CHUNK = 512
NBLK = CHUNK // BLK


def _make_gather(B, L):
    total = B * L
    per_w = total // NW
    n_chunk = per_w // CHUNK
    mesh = plsc.VectorSubcoreMesh(
        core_axis_name="c", subcore_axis_name="s",
        num_cores=NUM_CORES, num_subcores=NUM_SUBCORES)

    @functools.partial(
        pl.kernel,
        out_type=jax.ShapeDtypeStruct((L, EMB // 8, B // BLK, 8, BLK),
                                      jnp.float32),
        mesh=mesh,
        scratch_types=[
            [pltpu.VMEM((CHUNK,), jnp.int32) for _ in range(2)],
            [pltpu.VMEM((CHUNK, EMB), jnp.float32) for _ in range(2)],
            [pltpu.VMEM((EMB // 8, 8, BLK + 1), jnp.float32)
             for _ in range(2)],
            pltpu.SemaphoreType.DMA,
            pltpu.SemaphoreType.DMA,
            pltpu.SemaphoreType.DMA,
        ],
        compiler_params=pltpu.CompilerParams(
            use_tc_tiling_on_sc=False, needs_layout_passes=False),
    )
    def gather(ids_hbm, table_hbm, out_hbm, idx_v, rows_v, blk_v,
               idx_sem, gat_sem, out_sem):
        wid = lax.axis_index("s") * NUM_CORES + lax.axis_index("c")
        base = wid * per_w
        iota = lax.iota(jnp.int32, LANES)

        for s in range(2):
            pltpu.async_copy(
                ids_hbm.at[pl.ds(base + s * CHUNK, CHUNK)], idx_v[s], idx_sem)
        pltpu.make_async_copy(
            ids_hbm.at[pl.ds(base, CHUNK)], idx_v[0], idx_sem).wait()
        pltpu.async_copy(table_hbm.at[idx_v[0]], rows_v[0], gat_sem)

        d0s = [(jnp.arange(LANES, dtype=jnp.int32) + q * LANES) // 8
               for q in range(EMB // LANES)]
        d1s = [(jnp.arange(LANES, dtype=jnp.int32) + q * LANES) % 8
               for q in range(EMB // LANES)]

        def transpose_block(s, tb, k):
            def tbody(rr, c2):
                r0 = rr * LANES
                for half in range(2):
                    h0 = half * 8
                    vs = []
                    for ri in range(8):
                        row = k * BLK + r0 + h0 + ri
                        for q in range(EMB // LANES):
                            vs.append(
                                rows_v[s][row, pl.ds(q * LANES, LANES)])
                    for ri in range(8):
                        d2 = jnp.full((LANES,), h0 + ri, jnp.int32) + r0
                        for q in range(EMB // LANES):
                            plsc.store_scatter(
                                blk_v[tb], [d0s[q], d1s[q], d2],
                                vs[ri * (EMB // LANES) + q])
                return c2

            lax.fori_loop(0, BLK // LANES, tbody, 0)

        def body(g, carry):
            for s in range(2):
                i = 2 * g + s
                off = base + i * CHUNK
                @pl.when(i + 1 < n_chunk)
                def _():
                    pltpu.make_async_copy(
                        ids_hbm.at[pl.ds(base, CHUNK)], idx_v[1 - s],
                        idx_sem).wait()
                    pltpu.async_copy(
                        table_hbm.at[idx_v[1 - s]], rows_v[1 - s], gat_sem)
                pltpu.make_async_copy(
                    table_hbm.at[idx_v[s]], rows_v[s], gat_sem).wait()
                @pl.when(i + 2 < n_chunk)
                def _():
                    pltpu.async_copy(
                        ids_hbm.at[pl.ds(off + 2 * CHUNK, CHUNK)],
                        idx_v[s], idx_sem)

                l = lax.div(off, B)
                c0 = lax.div(lax.rem(off, B), BLK)
                for k in range(NBLK):
                    tb = k % 2
                    if k >= 2:
                        pltpu.make_async_copy(
                            blk_v[tb].at[:, :, pl.ds(0, BLK)],
                            out_hbm.at[0, :, 0, :, :], out_sem).wait()
                    else:
                        @pl.when(i > 0)
                        def _():
                            pltpu.make_async_copy(
                                blk_v[tb], out_hbm.at[0, :, 0, :, :],
                                out_sem).wait()
                    transpose_block(s, tb, k)
                    pltpu.async_copy(
                        blk_v[tb].at[:, :, pl.ds(0, BLK)],
                        out_hbm.at[l, :, c0 + k, :, :], out_sem)
            return carry

        lax.fori_loop(0, n_chunk // 2, body, 0)

        for s in range(2):
            pltpu.make_async_copy(
                blk_v[s].at[:, :, pl.ds(0, BLK)],
                out_hbm.at[0, :, 0, :, :], out_sem).wait()

    return gather


def kernel(token_ids, table):
    b, l = token_ids.shape
    ids_t = token_ids.T.reshape(-1)
    out5 = _make_gather(b, l)(ids_t, table)
    return out5.transpose(2, 4, 0, 1, 3).reshape(b, l, EMB)


# speedup vs baseline: 1.0363x; 1.0363x over previous
import functools

import jax
import jax.numpy as jnp
from jax import lax
from jax.experimental import pallas as pl
from jax.experimental.pallas import tpu as pltpu
from jax.experimental.pallas import tpu_sc as plsc

EMB = 64
NUM_CORES = 2
NUM_SUBCORES = 16
NW = NUM_CORES * NUM_SUBCORES
LANES = 16
BLK = 128
CHUNK = 512
NBLK = CHUNK // BLK


def _make_gather(B, L):
    total = B * L
    per_w = total // NW
    n_chunk = per_w // CHUNK
    mesh = plsc.VectorSubcoreMesh(
        core_axis_name="c", subcore_axis_name="s",
        num_cores=NUM_CORES, num_subcores=NUM_SUBCORES)

    @functools.partial(
        pl.kernel,
        out_type=jax.ShapeDtypeStruct((L, EMB // 8, B // BLK, 8, BLK),
                                      jnp.float32),
        mesh=mesh,
        scratch_types=[
            [pltpu.VMEM((CHUNK,), jnp.int32) for _ in range(2)],
            [pltpu.VMEM((CHUNK, EMB), jnp.float32) for _ in range(2)],
            [pltpu.VMEM((EMB // 8, 8, BLK + 1), jnp.float32)
             for _ in range(2)],
            pltpu.SemaphoreType.DMA,
            pltpu.SemaphoreType.DMA,
            pltpu.SemaphoreType.DMA,
        ],
        compiler_params=pltpu.CompilerParams(
            use_tc_tiling_on_sc=False, needs_layout_passes=False),
    )
    def gather(ids_hbm, table_hbm, out_hbm, idx_v, rows_v, blk_v,
               idx_sem, gat_sem, out_sem):
        wid = lax.axis_index("s") * NUM_CORES + lax.axis_index("c")
        base = wid * per_w
        iota = lax.iota(jnp.int32, LANES)

        for s in range(2):
            pltpu.async_copy(
                ids_hbm.at[pl.ds(base + s * CHUNK, CHUNK)], idx_v[s], idx_sem)
        pltpu.make_async_copy(
            ids_hbm.at[pl.ds(base, CHUNK)], idx_v[0], idx_sem).wait()
        pltpu.async_copy(table_hbm.at[idx_v[0]], rows_v[0], gat_sem)

        d0s = [(jnp.arange(LANES, dtype=jnp.int32) + q * LANES) // 8
               for q in range(EMB // LANES)]
        d1s = [(jnp.arange(LANES, dtype=jnp.int32) + q * LANES) % 8
               for q in range(EMB // LANES)]

        def transpose_block(s, tb, k):
            def tbody(rr, c2):
                r0 = rr * 8
                vs = []
                for ri in range(8):
                    row = k * BLK + r0 + ri
                    for q in range(EMB // LANES):
                        vs.append(rows_v[s][row, pl.ds(q * LANES, LANES)])
                for ri in range(8):
                    d2 = jnp.full((LANES,), ri, jnp.int32) + r0
                    for q in range(EMB // LANES):
                        plsc.store_scatter(
                            blk_v[tb], [d0s[q], d1s[q], d2],
                            vs[ri * (EMB // LANES) + q])
                return c2

            lax.fori_loop(0, BLK // 8, tbody, 0)

        def body(g, carry):
            for s in range(2):
                i = 2 * g + s
                off = base + i * CHUNK
                pltpu.make_async_copy(
                    table_hbm.at[idx_v[s]], rows_v[s], gat_sem).wait()
                @pl.when(i + 2 < n_chunk)
                def _():
                    pltpu.async_copy(
                        ids_hbm.at[pl.ds(off + 2 * CHUNK, CHUNK)],
                        idx_v[s], idx_sem)
                @pl.when(i + 1 < n_chunk)
                def _():
                    pltpu.make_async_copy(
                        ids_hbm.at[pl.ds(base, CHUNK)], idx_v[1 - s],
                        idx_sem).wait()
                    pltpu.async_copy(
                        table_hbm.at[idx_v[1 - s]], rows_v[1 - s], gat_sem)

                l = lax.div(off, B)
                c0 = lax.div(lax.rem(off, B), BLK)
                for k in range(NBLK):
                    tb = k % 2
                    if k >= 2:
                        pltpu.make_async_copy(
                            blk_v[tb].at[:, :, pl.ds(0, BLK)],
                            out_hbm.at[0, :, 0, :, :], out_sem).wait()
                    else:
                        @pl.when(i > 0)
                        def _():
                            pltpu.make_async_copy(
                                blk_v[tb], out_hbm.at[0, :, 0, :, :],
                                out_sem).wait()
                    transpose_block(s, tb, k)
                    pltpu.async_copy(
                        blk_v[tb].at[:, :, pl.ds(0, BLK)],
                        out_hbm.at[l, :, c0 + k, :, :], out_sem)
            return carry

        lax.fori_loop(0, n_chunk // 2, body, 0)

        for s in range(2):
            pltpu.make_async_copy(
                blk_v[s].at[:, :, pl.ds(0, BLK)],
                out_hbm.at[0, :, 0, :, :], out_sem).wait()

    return gather


def kernel(token_ids, table):
    b, l = token_ids.shape
    ids_t = token_ids.T.reshape(-1)
    out5 = _make_gather(b, l)(ids_t, table)
    return out5.transpose(2, 4, 0, 1, 3).reshape(b, l, EMB)
